# Initial kernel scaffold; baseline (speedup 1.0000x reference)
import jax
import jax.numpy as jnp
from jax.experimental import pallas as pl


def kernel(x, edge_index, u, batch, gamma_x, beta_x, gamma_u, beta_u, W1, b1, W2, b2, W3, b3, V1, c1, V2, c2, V3, c3):
    raise NotImplementedError("write your pallas kernel here")


# speedup vs baseline: 6.7907x; 6.7907x over previous
import functools

import jax
import jax.numpy as jnp
from jax import lax
from jax.experimental import pallas as pl
from jax.experimental.pallas import tpu as pltpu
from jax.experimental.pallas import tpu_sc as plsc

N = 10000
E = 320000
D = 128
H = 32
NB = 64
G = 2
BIGGER = 128
OUT = 1

NW = 32
EW = E // NW
C = 80
K = EW // C

N_PAD = 10240
RPT = N_PAD // 16
CW = 16

_MESH = dict(core_axis_name="c", subcore_axis_name="s")


def _tc_prep(x, gamma, beta, W1):
    def body(x_ref, g_ref, b_ref, w_ref, a_ref, bb_ref):
        xv = x_ref[...]
        m = jnp.mean(xv, axis=0, keepdims=True)
        v = jnp.mean((xv - m) ** 2, axis=0, keepdims=True)
        xn = (xv - m) * lax.rsqrt(v + 1e-5) * g_ref[...] + b_ref[...]
        wa = w_ref[:D, :]
        wb = w_ref[D:, :]
        a_ref[...] = jnp.dot(
            xn, wa - wb, preferred_element_type=jnp.float32
        ).astype(jnp.bfloat16)
        bb_ref[...] = jnp.dot(
            xn, wb, preferred_element_type=jnp.float32
        ).astype(jnp.bfloat16)

    return pl.pallas_call(
        body,
        out_shape=(jax.ShapeDtypeStruct((N, H), jnp.bfloat16),
                   jax.ShapeDtypeStruct((N, H), jnp.bfloat16)),
    )(x, gamma.reshape(1, D), beta.reshape(1, D), W1)


def _sc_gather(A, B, dst_r, src_r):
    @functools.partial(
        pl.kernel,
        out_type=(jax.ShapeDtypeStruct((E, H), jnp.bfloat16),
                  jax.ShapeDtypeStruct((E, H), jnp.bfloat16)),
        mesh=plsc.VectorSubcoreMesh(**_MESH),
        compiler_params=pltpu.CompilerParams(use_tc_tiling_on_sc=False),
        scratch_types=[
            pltpu.VMEM((K, C), jnp.int32),
            pltpu.VMEM((K, C), jnp.int32),
            pltpu.VMEM((C, H), jnp.bfloat16),
            pltpu.VMEM((C, H), jnp.bfloat16),
            pltpu.VMEM((C, H), jnp.bfloat16),
            pltpu.VMEM((C, H), jnp.bfloat16),
            pltpu.VMEM_SHARED((N, H), jnp.bfloat16),
            pltpu.VMEM_SHARED((N, H), jnp.bfloat16),
            pltpu.SemaphoreType.DMA,
            pltpu.SemaphoreType.DMA,
            pltpu.SemaphoreType.DMA,
            pltpu.SemaphoreType.DMA,
            pltpu.SemaphoreType.DMA,
            pltpu.SemaphoreType.DMA,
            pltpu.SemaphoreType.DMA,
            pltpu.SemaphoreType.DMA,
        ],
    )
    def k(a_hbm, b_hbm, dst_hbm, src_hbm, ga_hbm, gb_hbm,
          dst_v, src_v, buf_a0, buf_a1, buf_b0, buf_b1, a_sh, b_sh,
          sem_a0, sem_a1, sem_b0, sem_b1, osem_a0, osem_a1, osem_b0,
          osem_b1):
        sid = lax.axis_index("s")
        wid = lax.axis_index("c") * 16 + sid
        pltpu.sync_copy(dst_hbm.at[wid], dst_v)
        pltpu.sync_copy(src_hbm.at[wid], src_v)

        @pl.when(sid == 0)
        def _():
            pltpu.sync_copy(a_hbm, a_sh)
            pltpu.sync_copy(b_hbm, b_sh)

        plsc.subcore_barrier()
        base = wid * EW
        bufs_a = (buf_a0, buf_a1)
        bufs_b = (buf_b0, buf_b1)
        sems_a = (sem_a0, sem_a1)
        sems_b = (sem_b0, sem_b1)

        def start_g(j, b):
            pltpu.async_copy(a_sh.at[dst_v.at[j]], bufs_a[b], sems_a[b])
            pltpu.async_copy(b_sh.at[src_v.at[j]], bufs_b[b], sems_b[b])

        def wait_g(j, b):
            pltpu.make_async_copy(a_sh.at[dst_v.at[j]], bufs_a[b],
                                  sems_a[b]).wait()
            pltpu.make_async_copy(b_sh.at[src_v.at[j]], bufs_b[b],
                                  sems_b[b]).wait()

        osems_a = (osem_a0, osem_a1)
        osems_b = (osem_b0, osem_b1)

        def start_o(j, b):
            pltpu.async_copy(bufs_a[b], ga_hbm.at[pl.ds(base + j * C, C)],
                             osems_a[b])
            pltpu.async_copy(bufs_b[b], gb_hbm.at[pl.ds(base + j * C, C)],
                             osems_b[b])

        def wait_o(j, b):
            pltpu.make_async_copy(bufs_a[b],
                                  ga_hbm.at[pl.ds(base + j * C, C)],
                                  osems_a[b]).wait()
            pltpu.make_async_copy(bufs_b[b],
                                  gb_hbm.at[pl.ds(base + j * C, C)],
                                  osems_b[b]).wait()

        start_g(0, 0)
        wait_g(0, 0)
        start_o(0, 0)
        start_g(1, 1)

        def pair(jj, carry):
            j1 = 2 * jj + 1
            wait_g(j1, 1)
            start_o(j1, 1)
            wait_o(j1 - 1, 0)
            start_g(j1 + 1, 0)
            j2 = j1 + 1
            wait_g(j2, 0)
            start_o(j2, 0)
            wait_o(j2 - 1, 1)
            start_g(j2 + 1, 1)
            return carry

        lax.fori_loop(0, (K - 3) // 2, pair, 0)
        wait_g(K - 2, 1)
        start_o(K - 2, 1)
        wait_o(K - 3, 0)
        start_g(K - 1, 0)
        wait_g(K - 1, 0)
        start_o(K - 1, 0)
        wait_o(K - 2, 1)
        wait_o(K - 1, 0)

    return k(A, B, dst_r, src_r)


E4 = E // 4


def _tc_mlp(gA4, gB4, b1t, W2blk, b2t, W3blk, b3t):
    BE4 = 2000

    def body(a_ref, g_ref, b1_ref, w2_ref, b2_ref, w3_ref, b3_ref, o_ref):
        h = jnp.maximum(a_ref[...].astype(jnp.float32)
                        + g_ref[...].astype(jnp.float32) + b1_ref[...], 0.0)
        h = jnp.dot(h, w2_ref[...], preferred_element_type=jnp.float32)
        h = jnp.maximum(h + b2_ref[...], 0.0)
        h = jnp.dot(h, w3_ref[...], preferred_element_type=jnp.float32)
        o_ref[...] = jnp.maximum(h + b3_ref[...], 0.0)

    full = lambda i: (0, 0)
    return pl.pallas_call(
        body,
        grid=(E4 // BE4,),
        in_specs=[
            pl.BlockSpec((BE4, 128), lambda i: (i, 0)),
            pl.BlockSpec((BE4, 128), lambda i: (i, 0)),
            pl.BlockSpec((1, 128), full),
            pl.BlockSpec((128, 128), lambda i: (0, 0)),
            pl.BlockSpec((1, 128), full),
            pl.BlockSpec((128, 128), lambda i: (0, 0)),
            pl.BlockSpec((1, 128), full),
        ],
        out_specs=pl.BlockSpec((BE4, 128), lambda i: (i, 0)),
        out_shape=jax.ShapeDtypeStruct((E4, 128), jnp.float32),
    )(gA4, gB4, b1t, W2blk, b2t, W3blk, b3t)


def _sc_scatter(h3, dst_r, s_init, c_init, ones):
    @functools.partial(
        pl.kernel,
        out_type=(jax.ShapeDtypeStruct((2, N_PAD, H), jnp.float32),
                  jax.ShapeDtypeStruct((2, N_PAD, CW), jnp.float32)),
        mesh=plsc.VectorSubcoreMesh(**_MESH),
        compiler_params=pltpu.CompilerParams(use_tc_tiling_on_sc=False),
        scratch_types=[
            pltpu.VMEM((K, C), jnp.int32),
            pltpu.VMEM((C, H), jnp.float32),
            pltpu.VMEM((C, H), jnp.float32),
            pltpu.VMEM((C, CW), jnp.float32),
            pltpu.VMEM_SHARED((N_PAD, H), jnp.float32),
            pltpu.VMEM_SHARED((N_PAD, CW), jnp.float32),
            pltpu.SemaphoreType.DMA,
            pltpu.SemaphoreType.DMA,
            pltpu.SemaphoreType.DMA,
            pltpu.SemaphoreType.DMA,
            pltpu.SemaphoreType.DMA,
            pltpu.SemaphoreType.DMA,
        ],
    )
    def k(h3_hbm, dst_hbm, sinit_hbm, cinit_hbm, ones_hbm, s_hbm, cnt_hbm,
          dst_v, hbuf0, hbuf1, ones_v, s_acc, c_acc, sem0, sem1,
          ssem0, ssem1, csem0, csem1):
        cid = lax.axis_index("c")
        sid = lax.axis_index("s")
        wid = cid * 16 + sid
        pltpu.sync_copy(dst_hbm.at[wid], dst_v)
        pltpu.sync_copy(ones_hbm, ones_v)

        slab = pl.ds(sid * RPT, RPT)
        pltpu.sync_copy(sinit_hbm.at[slab], s_acc.at[slab])
        pltpu.sync_copy(cinit_hbm.at[slab], c_acc.at[slab])
        plsc.subcore_barrier()

        base = wid * EW
        hbufs = (hbuf0, hbuf1)
        sems = (sem0, sem1)

        def start_l(j, b):
            pltpu.async_copy(h3_hbm.at[pl.ds(base + j * C, C)], hbufs[b],
                             sems[b])

        def wait_l(j, b):
            pltpu.make_async_copy(h3_hbm.at[pl.ds(base + j * C, C)],
                                  hbufs[b], sems[b]).wait()

        ssems = (ssem0, ssem1)
        csems = (csem0, csem1)

        def start_s(j, b):
            pltpu.async_copy(hbufs[b], s_acc.at[dst_v.at[j]], ssems[b],
                             add=True)
            pltpu.async_copy(ones_v, c_acc.at[dst_v.at[j]], csems[b],
                             add=True)

        def wait_s(j, b):
            pltpu.make_async_copy(hbufs[b], s_acc.at[dst_v.at[j]],
                                  ssems[b]).wait()
            pltpu.make_async_copy(ones_v, c_acc.at[dst_v.at[j]],
                                  csems[b]).wait()

        start_l(0, 0)
        wait_l(0, 0)
        start_s(0, 0)
        start_l(1, 1)

        def pair(jj, carry):
            j1 = 2 * jj + 1
            wait_l(j1, 1)
            start_s(j1, 1)
            wait_s(j1 - 1, 0)
            start_l(j1 + 1, 0)
            j2 = j1 + 1
            wait_l(j2, 0)
            start_s(j2, 0)
            wait_s(j2 - 1, 1)
            start_l(j2 + 1, 1)
            return carry

        lax.fori_loop(0, (K - 3) // 2, pair, 0)
        wait_l(K - 2, 1)
        start_s(K - 2, 1)
        wait_s(K - 3, 0)
        start_l(K - 1, 0)
        wait_l(K - 1, 0)
        start_s(K - 1, 0)
        wait_s(K - 2, 1)
        wait_s(K - 1, 0)
        plsc.subcore_barrier()

        pltpu.sync_copy(s_acc.at[slab], s_hbm.at[cid, slab])
        pltpu.sync_copy(c_acc.at[slab], cnt_hbm.at[cid, slab])

    return k(h3, dst_r, s_init, c_init, ones)


def _tc_final(s_p, c_p, batch_pad, u, gamma_u, beta_u, V1, c1, V2, c2, V3, c3):
    def body(s_ref, c_ref, bt_ref, u_ref, gu_ref, bu_ref,
             v1_ref, c1_ref, v2_ref, c2_ref, v3_ref, c3_ref, o_ref):
        s = s_ref[0] + s_ref[1]
        cnt = c_ref[0, :, :1] + c_ref[1, :, :1]
        xc = s / jnp.maximum(cnt, 1.0)
        b = bt_ref[...]
        oh = (lax.broadcasted_iota(jnp.int32, (NB, N_PAD), 0) == b
              ).astype(jnp.float32)
        gs = jnp.dot(oh, xc, preferred_element_type=jnp.float32)
        gc = jnp.sum(oh, axis=1, keepdims=True)
        u2 = gs / jnp.maximum(gc, 1.0)
        uv = u_ref[...]
        mu = jnp.mean(uv, axis=0, keepdims=True)
        vu = jnp.mean((uv - mu) ** 2, axis=0, keepdims=True)
        u1 = (uv - mu) * lax.rsqrt(vu + 1e-5) * gu_ref[...] + bu_ref[...]
        uc = jnp.concatenate([u1, u2], axis=1)
        o = jnp.maximum(
            jnp.dot(uc, v1_ref[...], preferred_element_type=jnp.float32)
            + c1_ref[...], 0.0)
        o = jnp.maximum(
            jnp.dot(o, v2_ref[...], preferred_element_type=jnp.float32)
            + c2_ref[...], 0.0)
        o_ref[...] = (jnp.dot(o, v3_ref[...], preferred_element_type=jnp.float32)
                      + c3_ref[...])

    return pl.pallas_call(
        body,
        out_shape=jax.ShapeDtypeStruct((NB, OUT), jnp.float32),
    )(s_p, c_p, batch_pad, u, gamma_u.reshape(1, G), beta_u.reshape(1, G),
      V1, c1.reshape(1, BIGGER), V2, c2.reshape(1, BIGGER),
      V3, c3.reshape(1, OUT))


def kernel(x, edge_index, u, batch, gamma_x, beta_x, gamma_u, beta_u,
           W1, b1, W2, b2, W3, b3, V1, c1, V2, c2, V3, c3):
    src_r = edge_index[0].reshape(NW, K, C)
    dst_r = edge_index[1].reshape(NW, K, C)
    A, B = _tc_prep(x, gamma_x, beta_x, W1)
    gA, gB = _sc_gather(A, B, dst_r, src_r)
    eye4 = jnp.eye(4, dtype=jnp.float32)
    h3p = _tc_mlp(gA.reshape(E4, 128), gB.reshape(E4, 128),
                  jnp.tile(b1, 4).reshape(1, 128),
                  jnp.kron(eye4, W2), jnp.tile(b2, 4).reshape(1, 128),
                  jnp.kron(eye4, W3), jnp.tile(b3, 4).reshape(1, 128))
    s_p, c_p = _sc_scatter(h3p.reshape(E, H), dst_r,
                           jnp.zeros((N_PAD, H), jnp.float32),
                           jnp.zeros((N_PAD, CW), jnp.float32),
                           jnp.ones((C, CW), jnp.float32))
    batch_pad = jnp.concatenate(
        [batch, jnp.full((N_PAD - N,), NB, jnp.int32)]).reshape(1, N_PAD)
    return _tc_final(s_p, c_p, batch_pad, u, gamma_u, beta_u,
                     V1, c1, V2, c2, V3, c3)


# speedup vs baseline: 11.2217x; 1.6525x over previous
import functools

import jax
import jax.numpy as jnp
from jax import lax
from jax.experimental import pallas as pl
from jax.experimental.pallas import tpu as pltpu
from jax.experimental.pallas import tpu_sc as plsc

N = 10000
E = 320000
D = 128
H = 32
NB = 64
G = 2
BIGGER = 128
OUT = 1

NW = 32
EW = E // NW
C = 100
K = EW // C

N_PAD = 10240
RPT = N_PAD // 16
CW = 16

_MESH = dict(core_axis_name="c", subcore_axis_name="s")


def _tc_prep(x, gamma, beta, W1):
    def body(x_ref, g_ref, b_ref, w_ref, a_ref, bb_ref):
        xv = x_ref[...]
        m = jnp.mean(xv, axis=0, keepdims=True)
        v = jnp.mean((xv - m) ** 2, axis=0, keepdims=True)
        xn = (xv - m) * lax.rsqrt(v + 1e-5) * g_ref[...] + b_ref[...]
        wa = w_ref[:D, :]
        wb = w_ref[D:, :]
        a_ref[...] = jnp.dot(xn, wa - wb, preferred_element_type=jnp.float32)
        bb_ref[...] = jnp.dot(xn, wb, preferred_element_type=jnp.float32)

    return pl.pallas_call(
        body,
        out_shape=(jax.ShapeDtypeStruct((N, H), jnp.float32),
                   jax.ShapeDtypeStruct((N, H), jnp.float32)),
    )(x, gamma.reshape(1, D), beta.reshape(1, D), W1)


def _sc_gather(A, B, dst_r, src_r):
    @functools.partial(
        pl.kernel,
        out_type=(jax.ShapeDtypeStruct((E, H), jnp.float32),
                  jax.ShapeDtypeStruct((E, H), jnp.float32)),
        mesh=plsc.VectorSubcoreMesh(**_MESH),
        compiler_params=pltpu.CompilerParams(use_tc_tiling_on_sc=False),
        scratch_types=[
            pltpu.VMEM((K, C), jnp.int32),
            pltpu.VMEM((K, C), jnp.int32),
            pltpu.VMEM((C, H), jnp.float32),
            pltpu.VMEM((C, H), jnp.float32),
            pltpu.VMEM((C, H), jnp.float32),
            pltpu.VMEM((C, H), jnp.float32),
            pltpu.VMEM_SHARED((N, H), jnp.float32),
            pltpu.VMEM_SHARED((N, H), jnp.float32),
            pltpu.SemaphoreType.DMA,
            pltpu.SemaphoreType.DMA,
            pltpu.SemaphoreType.DMA,
            pltpu.SemaphoreType.DMA,
            pltpu.SemaphoreType.DMA,
            pltpu.SemaphoreType.DMA,
            pltpu.SemaphoreType.DMA,
            pltpu.SemaphoreType.DMA,
        ],
    )
    def k(a_hbm, b_hbm, dst_hbm, src_hbm, ga_hbm, gb_hbm,
          dst_v, src_v, buf_a0, buf_a1, buf_b0, buf_b1, a_sh, b_sh,
          sem_a0, sem_a1, sem_b0, sem_b1, osem_a0, osem_a1, osem_b0,
          osem_b1):
        sid = lax.axis_index("s")
        wid = lax.axis_index("c") * 16 + sid
        pltpu.sync_copy(dst_hbm.at[wid], dst_v)
        pltpu.sync_copy(src_hbm.at[wid], src_v)

        @pl.when(sid == 0)
        def _():
            pltpu.sync_copy(a_hbm, a_sh)
            pltpu.sync_copy(b_hbm, b_sh)

        plsc.subcore_barrier()
        base = wid * EW
        bufs_a = (buf_a0, buf_a1)
        bufs_b = (buf_b0, buf_b1)
        sems_a = (sem_a0, sem_a1)
        sems_b = (sem_b0, sem_b1)

        def start_g(j, b):
            pltpu.async_copy(a_sh.at[dst_v.at[j]], bufs_a[b], sems_a[b])
            pltpu.async_copy(b_sh.at[src_v.at[j]], bufs_b[b], sems_b[b])

        def wait_g(j, b):
            pltpu.make_async_copy(a_sh.at[dst_v.at[j]], bufs_a[b],
                                  sems_a[b]).wait()
            pltpu.make_async_copy(b_sh.at[src_v.at[j]], bufs_b[b],
                                  sems_b[b]).wait()

        osems_a = (osem_a0, osem_a1)
        osems_b = (osem_b0, osem_b1)

        def start_o(j, b):
            pltpu.async_copy(bufs_a[b], ga_hbm.at[pl.ds(base + j * C, C)],
                             osems_a[b])
            pltpu.async_copy(bufs_b[b], gb_hbm.at[pl.ds(base + j * C, C)],
                             osems_b[b])

        def wait_o(j, b):
            pltpu.make_async_copy(bufs_a[b],
                                  ga_hbm.at[pl.ds(base + j * C, C)],
                                  osems_a[b]).wait()
            pltpu.make_async_copy(bufs_b[b],
                                  gb_hbm.at[pl.ds(base + j * C, C)],
                                  osems_b[b]).wait()

        start_g(0, 0)
        wait_g(0, 0)
        start_o(0, 0)
        start_g(1, 1)

        def step(j, carry):
            for b in (0, 1):
                @pl.when(j % 2 == b)
                def _():
                    wait_g(j, b)
                    start_o(j, b)
                    wait_o(j - 1, 1 - b)
                    start_g(j + 1, 1 - b)
            return carry

        lax.fori_loop(1, K - 1, step, 0)
        bl = (K - 1) % 2
        wait_g(K - 1, bl)
        start_o(K - 1, bl)
        wait_o(K - 2, 1 - bl)
        wait_o(K - 1, bl)

    return k(A, B, dst_r, src_r)


E4 = E // 4


def _tc_mlp(gA4, gB4, b1t, W2blk, b2t, W3blk, b3t):
    BE4 = 2000

    def body(a_ref, g_ref, b1_ref, w2_ref, b2_ref, w3_ref, b3_ref, o_ref):
        h = jnp.maximum(a_ref[...] + g_ref[...] + b1_ref[...], 0.0)
        h = jnp.dot(h, w2_ref[...], preferred_element_type=jnp.float32)
        h = jnp.maximum(h + b2_ref[...], 0.0)
        h = jnp.dot(h, w3_ref[...], preferred_element_type=jnp.float32)
        o_ref[...] = jnp.maximum(h + b3_ref[...], 0.0)

    full = lambda i: (0, 0)
    return pl.pallas_call(
        body,
        grid=(E4 // BE4,),
        in_specs=[
            pl.BlockSpec((BE4, 128), lambda i: (i, 0)),
            pl.BlockSpec((BE4, 128), lambda i: (i, 0)),
            pl.BlockSpec((1, 128), full),
            pl.BlockSpec((128, 128), lambda i: (0, 0)),
            pl.BlockSpec((1, 128), full),
            pl.BlockSpec((128, 128), lambda i: (0, 0)),
            pl.BlockSpec((1, 128), full),
        ],
        out_specs=pl.BlockSpec((BE4, 128), lambda i: (i, 0)),
        out_shape=jax.ShapeDtypeStruct((E4, 128), jnp.float32),
    )(gA4, gB4, b1t, W2blk, b2t, W3blk, b3t)


def _sc_scatter(h3, dst_r, s_init, c_init, ones):
    @functools.partial(
        pl.kernel,
        out_type=(jax.ShapeDtypeStruct((2, N_PAD, H), jnp.float32),
                  jax.ShapeDtypeStruct((2, N_PAD, CW), jnp.float32)),
        mesh=plsc.VectorSubcoreMesh(**_MESH),
        compiler_params=pltpu.CompilerParams(use_tc_tiling_on_sc=False),
        scratch_types=[
            pltpu.VMEM((K, C), jnp.int32),
            pltpu.VMEM((C, H), jnp.float32),
            pltpu.VMEM((C, H), jnp.float32),
            pltpu.VMEM((C, CW), jnp.float32),
            pltpu.VMEM_SHARED((N_PAD, H), jnp.float32),
            pltpu.VMEM_SHARED((N_PAD, CW), jnp.float32),
            pltpu.SemaphoreType.DMA,
            pltpu.SemaphoreType.DMA,
            pltpu.SemaphoreType.DMA,
            pltpu.SemaphoreType.DMA,
            pltpu.SemaphoreType.DMA,
            pltpu.SemaphoreType.DMA,
        ],
    )
    def k(h3_hbm, dst_hbm, sinit_hbm, cinit_hbm, ones_hbm, s_hbm, cnt_hbm,
          dst_v, hbuf0, hbuf1, ones_v, s_acc, c_acc, sem0, sem1,
          ssem0, ssem1, csem0, csem1):
        cid = lax.axis_index("c")
        sid = lax.axis_index("s")
        wid = cid * 16 + sid
        pltpu.sync_copy(dst_hbm.at[wid], dst_v)
        pltpu.sync_copy(ones_hbm, ones_v)

        slab = pl.ds(sid * RPT, RPT)
        pltpu.sync_copy(sinit_hbm.at[slab], s_acc.at[slab])
        pltpu.sync_copy(cinit_hbm.at[slab], c_acc.at[slab])
        plsc.subcore_barrier()

        base = wid * EW
        hbufs = (hbuf0, hbuf1)
        sems = (sem0, sem1)

        def start_l(j, b):
            pltpu.async_copy(h3_hbm.at[pl.ds(base + j * C, C)], hbufs[b],
                             sems[b])

        def wait_l(j, b):
            pltpu.make_async_copy(h3_hbm.at[pl.ds(base + j * C, C)],
                                  hbufs[b], sems[b]).wait()

        ssems = (ssem0, ssem1)
        csems = (csem0, csem1)

        def start_s(j, b):
            pltpu.async_copy(hbufs[b], s_acc.at[dst_v.at[j]], ssems[b],
                             add=True)
            pltpu.async_copy(ones_v, c_acc.at[dst_v.at[j]], csems[b],
                             add=True)

        def wait_s(j, b):
            pltpu.make_async_copy(hbufs[b], s_acc.at[dst_v.at[j]],
                                  ssems[b]).wait()
            pltpu.make_async_copy(ones_v, c_acc.at[dst_v.at[j]],
                                  csems[b]).wait()

        start_l(0, 0)
        wait_l(0, 0)
        start_s(0, 0)
        start_l(1, 1)

        def step(j, carry):
            for b in (0, 1):
                @pl.when(j % 2 == b)
                def _():
                    wait_l(j, b)
                    start_s(j, b)
                    wait_s(j - 1, 1 - b)
                    start_l(j + 1, 1 - b)
            return carry

        lax.fori_loop(1, K - 1, step, 0)
        bl = (K - 1) % 2
        wait_l(K - 1, bl)
        start_s(K - 1, bl)
        wait_s(K - 2, 1 - bl)
        wait_s(K - 1, bl)
        plsc.subcore_barrier()

        pltpu.sync_copy(s_acc.at[slab], s_hbm.at[cid, slab])
        pltpu.sync_copy(c_acc.at[slab], cnt_hbm.at[cid, slab])

    return k(h3, dst_r, s_init, c_init, ones)


def _tc_final(s_p, c_p, batch_pad, u, gamma_u, beta_u, V1, c1, V2, c2, V3, c3):
    def body(s_ref, c_ref, bt_ref, u_ref, gu_ref, bu_ref,
             v1_ref, c1_ref, v2_ref, c2_ref, v3_ref, c3_ref, o_ref):
        s = s_ref[0] + s_ref[1]
        cnt = c_ref[0, :, :1] + c_ref[1, :, :1]
        xc = s / jnp.maximum(cnt, 1.0)
        b = bt_ref[...]
        oh = (lax.broadcasted_iota(jnp.int32, (NB, N_PAD), 0) == b
              ).astype(jnp.float32)
        gs = jnp.dot(oh, xc, preferred_element_type=jnp.float32)
        gc = jnp.sum(oh, axis=1, keepdims=True)
        u2 = gs / jnp.maximum(gc, 1.0)
        uv = u_ref[...]
        mu = jnp.mean(uv, axis=0, keepdims=True)
        vu = jnp.mean((uv - mu) ** 2, axis=0, keepdims=True)
        u1 = (uv - mu) * lax.rsqrt(vu + 1e-5) * gu_ref[...] + bu_ref[...]
        uc = jnp.concatenate([u1, u2], axis=1)
        o = jnp.maximum(
            jnp.dot(uc, v1_ref[...], preferred_element_type=jnp.float32)
            + c1_ref[...], 0.0)
        o = jnp.maximum(
            jnp.dot(o, v2_ref[...], preferred_element_type=jnp.float32)
            + c2_ref[...], 0.0)
        o_ref[...] = (jnp.dot(o, v3_ref[...], preferred_element_type=jnp.float32)
                      + c3_ref[...])

    return pl.pallas_call(
        body,
        out_shape=jax.ShapeDtypeStruct((NB, OUT), jnp.float32),
    )(s_p, c_p, batch_pad, u, gamma_u.reshape(1, G), beta_u.reshape(1, G),
      V1, c1.reshape(1, BIGGER), V2, c2.reshape(1, BIGGER),
      V3, c3.reshape(1, OUT))


def kernel(x, edge_index, u, batch, gamma_x, beta_x, gamma_u, beta_u,
           W1, b1, W2, b2, W3, b3, V1, c1, V2, c2, V3, c3):
    src_r = edge_index[0].reshape(NW, K, C)
    dst_r = edge_index[1].reshape(NW, K, C)
    A, B = _tc_prep(x, gamma_x, beta_x, W1)
    gA, gB = _sc_gather(A, B, dst_r, src_r)
    eye4 = jnp.eye(4, dtype=jnp.float32)
    h3p = _tc_mlp(gA.reshape(E4, 128), gB.reshape(E4, 128),
                  jnp.tile(b1, 4).reshape(1, 128),
                  jnp.kron(eye4, W2), jnp.tile(b2, 4).reshape(1, 128),
                  jnp.kron(eye4, W3), jnp.tile(b3, 4).reshape(1, 128))
    s_p, c_p = _sc_scatter(h3p.reshape(E, H), dst_r,
                           jnp.zeros((N_PAD, H), jnp.float32),
                           jnp.zeros((N_PAD, CW), jnp.float32),
                           jnp.ones((C, CW), jnp.float32))
    batch_pad = jnp.concatenate(
        [batch, jnp.full((N_PAD - N,), NB, jnp.int32)]).reshape(1, N_PAD)
    return _tc_final(s_p, c_p, batch_pad, u, gamma_u, beta_u,
                     V1, c1, V2, c2, V3, c3)


# speedup vs baseline: 11.7936x; 1.0510x over previous
import functools

import jax
import jax.numpy as jnp
from jax import lax
from jax.experimental import pallas as pl
from jax.experimental.pallas import tpu as pltpu
from jax.experimental.pallas import tpu_sc as plsc

N = 10000
E = 320000
D = 128
H = 32
NB = 64
G = 2
BIGGER = 128
OUT = 1

NW = 32
EW = E // NW
C = 125
K = EW // C

N_PAD = 10240
RPT = N_PAD // 16
CW = 16

_MESH = dict(core_axis_name="c", subcore_axis_name="s")


def _tc_prep(x, gamma, beta, W1):
    def body(x_ref, g_ref, b_ref, w_ref, a_ref, bb_ref):
        xv = x_ref[...]
        m = jnp.mean(xv, axis=0, keepdims=True)
        v = jnp.mean((xv - m) ** 2, axis=0, keepdims=True)
        xn = (xv - m) * lax.rsqrt(v + 1e-5) * g_ref[...] + b_ref[...]
        wa = w_ref[:D, :]
        wb = w_ref[D:, :]
        a_ref[...] = jnp.dot(xn, wa - wb, preferred_element_type=jnp.float32)
        bb_ref[...] = jnp.dot(xn, wb, preferred_element_type=jnp.float32)

    return pl.pallas_call(
        body,
        out_shape=(jax.ShapeDtypeStruct((N, H), jnp.float32),
                   jax.ShapeDtypeStruct((N, H), jnp.float32)),
    )(x, gamma.reshape(1, D), beta.reshape(1, D), W1)


def _sc_gather(A, B, dst_r, src_r):
    @functools.partial(
        pl.kernel,
        out_type=(jax.ShapeDtypeStruct((E, H), jnp.float32),
                  jax.ShapeDtypeStruct((E, H), jnp.float32)),
        mesh=plsc.VectorSubcoreMesh(**_MESH),
        compiler_params=pltpu.CompilerParams(use_tc_tiling_on_sc=False),
        scratch_types=[
            pltpu.VMEM((K, C), jnp.int32),
            pltpu.VMEM((K, C), jnp.int32),
            pltpu.VMEM((C, H), jnp.float32),
            pltpu.VMEM((C, H), jnp.float32),
            pltpu.VMEM((C, H), jnp.float32),
            pltpu.VMEM((C, H), jnp.float32),
            pltpu.VMEM_SHARED((N, H), jnp.float32),
            pltpu.VMEM_SHARED((N, H), jnp.float32),
            pltpu.SemaphoreType.DMA,
            pltpu.SemaphoreType.DMA,
            pltpu.SemaphoreType.DMA,
            pltpu.SemaphoreType.DMA,
            pltpu.SemaphoreType.DMA,
            pltpu.SemaphoreType.DMA,
            pltpu.SemaphoreType.DMA,
            pltpu.SemaphoreType.DMA,
        ],
    )
    def k(a_hbm, b_hbm, dst_hbm, src_hbm, ga_hbm, gb_hbm,
          dst_v, src_v, buf_a0, buf_a1, buf_b0, buf_b1, a_sh, b_sh,
          sem_a0, sem_a1, sem_b0, sem_b1, osem_a0, osem_a1, osem_b0,
          osem_b1):
        sid = lax.axis_index("s")
        wid = lax.axis_index("c") * 16 + sid
        pltpu.sync_copy(dst_hbm.at[wid], dst_v)
        pltpu.sync_copy(src_hbm.at[wid], src_v)

        @pl.when(sid == 0)
        def _():
            pltpu.sync_copy(a_hbm, a_sh)
            pltpu.sync_copy(b_hbm, b_sh)

        plsc.subcore_barrier()
        base = wid * EW
        bufs_a = (buf_a0, buf_a1)
        bufs_b = (buf_b0, buf_b1)
        sems_a = (sem_a0, sem_a1)
        sems_b = (sem_b0, sem_b1)

        def start_g(j, b):
            pltpu.async_copy(a_sh.at[dst_v.at[j]], bufs_a[b], sems_a[b])
            pltpu.async_copy(b_sh.at[src_v.at[j]], bufs_b[b], sems_b[b])

        def wait_g(j, b):
            pltpu.make_async_copy(a_sh.at[dst_v.at[j]], bufs_a[b],
                                  sems_a[b]).wait()
            pltpu.make_async_copy(b_sh.at[src_v.at[j]], bufs_b[b],
                                  sems_b[b]).wait()

        osems_a = (osem_a0, osem_a1)
        osems_b = (osem_b0, osem_b1)

        def start_o(j, b):
            pltpu.async_copy(bufs_a[b], ga_hbm.at[pl.ds(base + j * C, C)],
                             osems_a[b])
            pltpu.async_copy(bufs_b[b], gb_hbm.at[pl.ds(base + j * C, C)],
                             osems_b[b])

        def wait_o(j, b):
            pltpu.make_async_copy(bufs_a[b],
                                  ga_hbm.at[pl.ds(base + j * C, C)],
                                  osems_a[b]).wait()
            pltpu.make_async_copy(bufs_b[b],
                                  gb_hbm.at[pl.ds(base + j * C, C)],
                                  osems_b[b]).wait()

        start_g(0, 0)
        wait_g(0, 0)
        start_o(0, 0)
        start_g(1, 1)

        def step(j, carry):
            for b in (0, 1):
                @pl.when(j % 2 == b)
                def _():
                    wait_g(j, b)
                    start_o(j, b)
                    wait_o(j - 1, 1 - b)
                    start_g(j + 1, 1 - b)
            return carry

        lax.fori_loop(1, K - 1, step, 0)
        bl = (K - 1) % 2
        wait_g(K - 1, bl)
        start_o(K - 1, bl)
        wait_o(K - 2, 1 - bl)
        wait_o(K - 1, bl)

    return k(A, B, dst_r, src_r)


E4 = E // 4


def _tc_mlp(gA4, gB4, b1t, W2blk, b2t, W3blk, b3t):
    BE4 = 2000

    def body(a_ref, g_ref, b1_ref, w2_ref, b2_ref, w3_ref, b3_ref, o_ref):
        h = jnp.maximum(a_ref[...] + g_ref[...] + b1_ref[...], 0.0)
        h = jnp.dot(h, w2_ref[...], preferred_element_type=jnp.float32)
        h = jnp.maximum(h + b2_ref[...], 0.0)
        h = jnp.dot(h, w3_ref[...], preferred_element_type=jnp.float32)
        o_ref[...] = jnp.maximum(h + b3_ref[...], 0.0)

    full = lambda i: (0, 0)
    return pl.pallas_call(
        body,
        grid=(E4 // BE4,),
        in_specs=[
            pl.BlockSpec((BE4, 128), lambda i: (i, 0)),
            pl.BlockSpec((BE4, 128), lambda i: (i, 0)),
            pl.BlockSpec((1, 128), full),
            pl.BlockSpec((128, 128), lambda i: (0, 0)),
            pl.BlockSpec((1, 128), full),
            pl.BlockSpec((128, 128), lambda i: (0, 0)),
            pl.BlockSpec((1, 128), full),
        ],
        out_specs=pl.BlockSpec((BE4, 128), lambda i: (i, 0)),
        out_shape=jax.ShapeDtypeStruct((E4, 128), jnp.float32),
    )(gA4, gB4, b1t, W2blk, b2t, W3blk, b3t)


def _sc_scatter(h3, dst_r, s_init, c_init, ones):
    @functools.partial(
        pl.kernel,
        out_type=(jax.ShapeDtypeStruct((2, N_PAD, H), jnp.float32),
                  jax.ShapeDtypeStruct((2, N_PAD, CW), jnp.float32)),
        mesh=plsc.VectorSubcoreMesh(**_MESH),
        compiler_params=pltpu.CompilerParams(use_tc_tiling_on_sc=False),
        scratch_types=[
            pltpu.VMEM((K, C), jnp.int32),
            pltpu.VMEM((C, H), jnp.float32),
            pltpu.VMEM((C, H), jnp.float32),
            pltpu.VMEM((C, CW), jnp.float32),
            pltpu.VMEM_SHARED((N_PAD, H), jnp.float32),
            pltpu.VMEM_SHARED((N_PAD, CW), jnp.float32),
            pltpu.SemaphoreType.DMA,
            pltpu.SemaphoreType.DMA,
            pltpu.SemaphoreType.DMA,
            pltpu.SemaphoreType.DMA,
            pltpu.SemaphoreType.DMA,
            pltpu.SemaphoreType.DMA,
        ],
    )
    def k(h3_hbm, dst_hbm, sinit_hbm, cinit_hbm, ones_hbm, s_hbm, cnt_hbm,
          dst_v, hbuf0, hbuf1, ones_v, s_acc, c_acc, sem0, sem1,
          ssem0, ssem1, csem0, csem1):
        cid = lax.axis_index("c")
        sid = lax.axis_index("s")
        wid = cid * 16 + sid
        pltpu.sync_copy(dst_hbm.at[wid], dst_v)
        pltpu.sync_copy(ones_hbm, ones_v)

        slab = pl.ds(sid * RPT, RPT)
        pltpu.sync_copy(sinit_hbm.at[slab], s_acc.at[slab])
        pltpu.sync_copy(cinit_hbm.at[slab], c_acc.at[slab])
        plsc.subcore_barrier()

        base = wid * EW
        hbufs = (hbuf0, hbuf1)
        sems = (sem0, sem1)

        def start_l(j, b):
            pltpu.async_copy(h3_hbm.at[pl.ds(base + j * C, C)], hbufs[b],
                             sems[b])

        def wait_l(j, b):
            pltpu.make_async_copy(h3_hbm.at[pl.ds(base + j * C, C)],
                                  hbufs[b], sems[b]).wait()

        ssems = (ssem0, ssem1)
        csems = (csem0, csem1)

        def start_s(j, b):
            pltpu.async_copy(hbufs[b], s_acc.at[dst_v.at[j]], ssems[b],
                             add=True)
            pltpu.async_copy(ones_v, c_acc.at[dst_v.at[j]], csems[b],
                             add=True)

        def wait_s(j, b):
            pltpu.make_async_copy(hbufs[b], s_acc.at[dst_v.at[j]],
                                  ssems[b]).wait()
            pltpu.make_async_copy(ones_v, c_acc.at[dst_v.at[j]],
                                  csems[b]).wait()

        start_l(0, 0)
        wait_l(0, 0)
        start_s(0, 0)
        start_l(1, 1)

        def step(j, carry):
            for b in (0, 1):
                @pl.when(j % 2 == b)
                def _():
                    wait_l(j, b)
                    start_s(j, b)
                    wait_s(j - 1, 1 - b)
                    start_l(j + 1, 1 - b)
            return carry

        lax.fori_loop(1, K - 1, step, 0)
        bl = (K - 1) % 2
        wait_l(K - 1, bl)
        start_s(K - 1, bl)
        wait_s(K - 2, 1 - bl)
        wait_s(K - 1, bl)
        plsc.subcore_barrier()

        pltpu.sync_copy(s_acc.at[slab], s_hbm.at[cid, slab])
        pltpu.sync_copy(c_acc.at[slab], cnt_hbm.at[cid, slab])

    return k(h3, dst_r, s_init, c_init, ones)


def _tc_final(s_p, c_p, batch_pad, u, gamma_u, beta_u, V1, c1, V2, c2, V3, c3):
    def body(s_ref, c_ref, bt_ref, u_ref, gu_ref, bu_ref,
             v1_ref, c1_ref, v2_ref, c2_ref, v3_ref, c3_ref, o_ref):
        s = s_ref[0] + s_ref[1]
        cnt = c_ref[0, :, :1] + c_ref[1, :, :1]
        xc = s / jnp.maximum(cnt, 1.0)
        b = bt_ref[...]
        oh = (lax.broadcasted_iota(jnp.int32, (NB, N_PAD), 0) == b
              ).astype(jnp.float32)
        gs = jnp.dot(oh, xc, preferred_element_type=jnp.float32)
        gc = jnp.sum(oh, axis=1, keepdims=True)
        u2 = gs / jnp.maximum(gc, 1.0)
        uv = u_ref[...]
        mu = jnp.mean(uv, axis=0, keepdims=True)
        vu = jnp.mean((uv - mu) ** 2, axis=0, keepdims=True)
        u1 = (uv - mu) * lax.rsqrt(vu + 1e-5) * gu_ref[...] + bu_ref[...]
        uc = jnp.concatenate([u1, u2], axis=1)
        o = jnp.maximum(
            jnp.dot(uc, v1_ref[...], preferred_element_type=jnp.float32)
            + c1_ref[...], 0.0)
        o = jnp.maximum(
            jnp.dot(o, v2_ref[...], preferred_element_type=jnp.float32)
            + c2_ref[...], 0.0)
        o_ref[...] = (jnp.dot(o, v3_ref[...], preferred_element_type=jnp.float32)
                      + c3_ref[...])

    return pl.pallas_call(
        body,
        out_shape=jax.ShapeDtypeStruct((NB, OUT), jnp.float32),
    )(s_p, c_p, batch_pad, u, gamma_u.reshape(1, G), beta_u.reshape(1, G),
      V1, c1.reshape(1, BIGGER), V2, c2.reshape(1, BIGGER),
      V3, c3.reshape(1, OUT))


def kernel(x, edge_index, u, batch, gamma_x, beta_x, gamma_u, beta_u,
           W1, b1, W2, b2, W3, b3, V1, c1, V2, c2, V3, c3):
    src_r = edge_index[0].reshape(NW, K, C)
    dst_r = edge_index[1].reshape(NW, K, C)
    A, B = _tc_prep(x, gamma_x, beta_x, W1)
    gA, gB = _sc_gather(A, B, dst_r, src_r)
    eye4 = jnp.eye(4, dtype=jnp.float32)
    h3p = _tc_mlp(gA.reshape(E4, 128), gB.reshape(E4, 128),
                  jnp.tile(b1, 4).reshape(1, 128),
                  jnp.kron(eye4, W2), jnp.tile(b2, 4).reshape(1, 128),
                  jnp.kron(eye4, W3), jnp.tile(b3, 4).reshape(1, 128))
    s_p, c_p = _sc_scatter(h3p.reshape(E, H), dst_r,
                           jnp.zeros((N_PAD, H), jnp.float32),
                           jnp.zeros((N_PAD, CW), jnp.float32),
                           jnp.ones((C, CW), jnp.float32))
    batch_pad = jnp.concatenate(
        [batch, jnp.full((N_PAD - N,), NB, jnp.int32)]).reshape(1, N_PAD)
    return _tc_final(s_p, c_p, batch_pad, u, gamma_u, beta_u,
                     V1, c1, V2, c2, V3, c3)


# speedup vs baseline: 13.6351x; 1.1561x over previous
import functools

import jax
import jax.numpy as jnp
from jax import lax
from jax.experimental import pallas as pl
from jax.experimental.pallas import tpu as pltpu
from jax.experimental.pallas import tpu_sc as plsc

N = 10000
E = 320000
D = 128
H = 32
NB = 64
G = 2
BIGGER = 128
OUT = 1

NW = 32
EW = E // NW
C = 500
K = EW // C

N_PAD = 10240
RPT = N_PAD // 16
CW = 16

_MESH = dict(core_axis_name="c", subcore_axis_name="s")


def _tc_prep(x, gamma, beta, W1):
    def body(x_ref, g_ref, b_ref, w_ref, a_ref, bb_ref):
        xv = x_ref[...]
        m = jnp.mean(xv, axis=0, keepdims=True)
        v = jnp.mean((xv - m) ** 2, axis=0, keepdims=True)
        xn = (xv - m) * lax.rsqrt(v + 1e-5) * g_ref[...] + b_ref[...]
        wa = w_ref[:D, :]
        wb = w_ref[D:, :]
        a_ref[...] = jnp.dot(xn, wa - wb, preferred_element_type=jnp.float32)
        bb_ref[...] = jnp.dot(xn, wb, preferred_element_type=jnp.float32)

    return pl.pallas_call(
        body,
        out_shape=(jax.ShapeDtypeStruct((N, H), jnp.float32),
                   jax.ShapeDtypeStruct((N, H), jnp.float32)),
    )(x, gamma.reshape(1, D), beta.reshape(1, D), W1)


def _sc_gather(A, B, dst_r, src_r):
    @functools.partial(
        pl.kernel,
        out_type=(jax.ShapeDtypeStruct((E, H), jnp.float32),
                  jax.ShapeDtypeStruct((E, H), jnp.float32)),
        mesh=plsc.VectorSubcoreMesh(**_MESH),
        compiler_params=pltpu.CompilerParams(use_tc_tiling_on_sc=False),
        scratch_types=[
            pltpu.VMEM((K, C), jnp.int32),
            pltpu.VMEM((K, C), jnp.int32),
            pltpu.VMEM((C, H), jnp.float32),
            pltpu.VMEM((C, H), jnp.float32),
            pltpu.VMEM((C, H), jnp.float32),
            pltpu.VMEM((C, H), jnp.float32),
            pltpu.VMEM_SHARED((N, H), jnp.float32),
            pltpu.VMEM_SHARED((N, H), jnp.float32),
            pltpu.SemaphoreType.DMA,
            pltpu.SemaphoreType.DMA,
            pltpu.SemaphoreType.DMA,
            pltpu.SemaphoreType.DMA,
            pltpu.SemaphoreType.DMA,
            pltpu.SemaphoreType.DMA,
            pltpu.SemaphoreType.DMA,
            pltpu.SemaphoreType.DMA,
        ],
    )
    def k(a_hbm, b_hbm, dst_hbm, src_hbm, ga_hbm, gb_hbm,
          dst_v, src_v, buf_a0, buf_a1, buf_b0, buf_b1, a_sh, b_sh,
          sem_a0, sem_a1, sem_b0, sem_b1, osem_a0, osem_a1, osem_b0,
          osem_b1):
        sid = lax.axis_index("s")
        wid = lax.axis_index("c") * 16 + sid
        pltpu.sync_copy(dst_hbm.at[wid], dst_v)
        pltpu.sync_copy(src_hbm.at[wid], src_v)

        @pl.when(sid == 0)
        def _():
            pltpu.sync_copy(a_hbm, a_sh)
            pltpu.sync_copy(b_hbm, b_sh)

        plsc.subcore_barrier()
        base = wid * EW
        bufs_a = (buf_a0, buf_a1)
        bufs_b = (buf_b0, buf_b1)
        sems_a = (sem_a0, sem_a1)
        sems_b = (sem_b0, sem_b1)

        def start_g(j, b):
            pltpu.async_copy(a_sh.at[dst_v.at[j]], bufs_a[b], sems_a[b])
            pltpu.async_copy(b_sh.at[src_v.at[j]], bufs_b[b], sems_b[b])

        def wait_g(j, b):
            pltpu.make_async_copy(a_sh.at[dst_v.at[j]], bufs_a[b],
                                  sems_a[b]).wait()
            pltpu.make_async_copy(b_sh.at[src_v.at[j]], bufs_b[b],
                                  sems_b[b]).wait()

        osems_a = (osem_a0, osem_a1)
        osems_b = (osem_b0, osem_b1)

        def start_o(j, b):
            pltpu.async_copy(bufs_a[b], ga_hbm.at[pl.ds(base + j * C, C)],
                             osems_a[b])
            pltpu.async_copy(bufs_b[b], gb_hbm.at[pl.ds(base + j * C, C)],
                             osems_b[b])

        def wait_o(j, b):
            pltpu.make_async_copy(bufs_a[b],
                                  ga_hbm.at[pl.ds(base + j * C, C)],
                                  osems_a[b]).wait()
            pltpu.make_async_copy(bufs_b[b],
                                  gb_hbm.at[pl.ds(base + j * C, C)],
                                  osems_b[b]).wait()

        start_g(0, 0)
        wait_g(0, 0)
        start_o(0, 0)
        start_g(1, 1)

        def step(j, carry):
            for b in (0, 1):
                @pl.when(j % 2 == b)
                def _():
                    wait_g(j, b)
                    start_o(j, b)
                    wait_o(j - 1, 1 - b)
                    start_g(j + 1, 1 - b)
            return carry

        lax.fori_loop(1, K - 1, step, 0)
        bl = (K - 1) % 2
        wait_g(K - 1, bl)
        start_o(K - 1, bl)
        wait_o(K - 2, 1 - bl)
        wait_o(K - 1, bl)

    return k(A, B, dst_r, src_r)


E4 = E // 4


def _tc_mlp(gA4, gB4, b1t, W2blk, b2t, W3blk, b3t):
    BE4 = 2000

    def body(a_ref, g_ref, b1_ref, w2_ref, b2_ref, w3_ref, b3_ref, o_ref):
        h = jnp.maximum(a_ref[...] + g_ref[...] + b1_ref[...], 0.0)
        h = jnp.dot(h, w2_ref[...], preferred_element_type=jnp.float32)
        h = jnp.maximum(h + b2_ref[...], 0.0)
        h = jnp.dot(h, w3_ref[...], preferred_element_type=jnp.float32)
        o_ref[...] = jnp.maximum(h + b3_ref[...], 0.0)

    full = lambda i: (0, 0)
    return pl.pallas_call(
        body,
        grid=(E4 // BE4,),
        in_specs=[
            pl.BlockSpec((BE4, 128), lambda i: (i, 0)),
            pl.BlockSpec((BE4, 128), lambda i: (i, 0)),
            pl.BlockSpec((1, 128), full),
            pl.BlockSpec((128, 128), lambda i: (0, 0)),
            pl.BlockSpec((1, 128), full),
            pl.BlockSpec((128, 128), lambda i: (0, 0)),
            pl.BlockSpec((1, 128), full),
        ],
        out_specs=pl.BlockSpec((BE4, 128), lambda i: (i, 0)),
        out_shape=jax.ShapeDtypeStruct((E4, 128), jnp.float32),
    )(gA4, gB4, b1t, W2blk, b2t, W3blk, b3t)


def _sc_scatter(h3, dst_r, s_init, c_init, ones):
    @functools.partial(
        pl.kernel,
        out_type=(jax.ShapeDtypeStruct((2, N_PAD, H), jnp.float32),
                  jax.ShapeDtypeStruct((2, N_PAD, CW), jnp.float32)),
        mesh=plsc.VectorSubcoreMesh(**_MESH),
        compiler_params=pltpu.CompilerParams(use_tc_tiling_on_sc=False),
        scratch_types=[
            pltpu.VMEM((K, C), jnp.int32),
            pltpu.VMEM((C, H), jnp.float32),
            pltpu.VMEM((C, H), jnp.float32),
            pltpu.VMEM((C, CW), jnp.float32),
            pltpu.VMEM_SHARED((N_PAD, H), jnp.float32),
            pltpu.VMEM_SHARED((N_PAD, CW), jnp.float32),
            pltpu.SemaphoreType.DMA,
            pltpu.SemaphoreType.DMA,
            pltpu.SemaphoreType.DMA,
            pltpu.SemaphoreType.DMA,
            pltpu.SemaphoreType.DMA,
            pltpu.SemaphoreType.DMA,
        ],
    )
    def k(h3_hbm, dst_hbm, sinit_hbm, cinit_hbm, ones_hbm, s_hbm, cnt_hbm,
          dst_v, hbuf0, hbuf1, ones_v, s_acc, c_acc, sem0, sem1,
          ssem0, ssem1, csem0, csem1):
        cid = lax.axis_index("c")
        sid = lax.axis_index("s")
        wid = cid * 16 + sid
        pltpu.sync_copy(dst_hbm.at[wid], dst_v)
        pltpu.sync_copy(ones_hbm, ones_v)

        slab = pl.ds(sid * RPT, RPT)
        pltpu.sync_copy(sinit_hbm.at[slab], s_acc.at[slab])
        pltpu.sync_copy(cinit_hbm.at[slab], c_acc.at[slab])
        plsc.subcore_barrier()

        base = wid * EW
        hbufs = (hbuf0, hbuf1)
        sems = (sem0, sem1)

        def start_l(j, b):
            pltpu.async_copy(h3_hbm.at[pl.ds(base + j * C, C)], hbufs[b],
                             sems[b])

        def wait_l(j, b):
            pltpu.make_async_copy(h3_hbm.at[pl.ds(base + j * C, C)],
                                  hbufs[b], sems[b]).wait()

        ssems = (ssem0, ssem1)
        csems = (csem0, csem1)

        def start_s(j, b):
            pltpu.async_copy(hbufs[b], s_acc.at[dst_v.at[j]], ssems[b],
                             add=True)
            pltpu.async_copy(ones_v, c_acc.at[dst_v.at[j]], csems[b],
                             add=True)

        def wait_s(j, b):
            pltpu.make_async_copy(hbufs[b], s_acc.at[dst_v.at[j]],
                                  ssems[b]).wait()
            pltpu.make_async_copy(ones_v, c_acc.at[dst_v.at[j]],
                                  csems[b]).wait()

        start_l(0, 0)
        wait_l(0, 0)
        start_s(0, 0)
        start_l(1, 1)

        def step(j, carry):
            for b in (0, 1):
                @pl.when(j % 2 == b)
                def _():
                    wait_l(j, b)
                    start_s(j, b)
                    wait_s(j - 1, 1 - b)
                    start_l(j + 1, 1 - b)
            return carry

        lax.fori_loop(1, K - 1, step, 0)
        bl = (K - 1) % 2
        wait_l(K - 1, bl)
        start_s(K - 1, bl)
        wait_s(K - 2, 1 - bl)
        wait_s(K - 1, bl)
        plsc.subcore_barrier()

        pltpu.sync_copy(s_acc.at[slab], s_hbm.at[cid, slab])
        pltpu.sync_copy(c_acc.at[slab], cnt_hbm.at[cid, slab])

    return k(h3, dst_r, s_init, c_init, ones)


def _tc_final(s_p, c_p, batch_pad, u, gamma_u, beta_u, V1, c1, V2, c2, V3, c3):
    def body(s_ref, c_ref, bt_ref, u_ref, gu_ref, bu_ref,
             v1_ref, c1_ref, v2_ref, c2_ref, v3_ref, c3_ref, o_ref):
        s = s_ref[0] + s_ref[1]
        cnt = c_ref[0, :, :1] + c_ref[1, :, :1]
        xc = s / jnp.maximum(cnt, 1.0)
        b = bt_ref[...]
        oh = (lax.broadcasted_iota(jnp.int32, (NB, N_PAD), 0) == b
              ).astype(jnp.float32)
        gs = jnp.dot(oh, xc, preferred_element_type=jnp.float32)
        gc = jnp.sum(oh, axis=1, keepdims=True)
        u2 = gs / jnp.maximum(gc, 1.0)
        uv = u_ref[...]
        mu = jnp.mean(uv, axis=0, keepdims=True)
        vu = jnp.mean((uv - mu) ** 2, axis=0, keepdims=True)
        u1 = (uv - mu) * lax.rsqrt(vu + 1e-5) * gu_ref[...] + bu_ref[...]
        uc = jnp.concatenate([u1, u2], axis=1)
        o = jnp.maximum(
            jnp.dot(uc, v1_ref[...], preferred_element_type=jnp.float32)
            + c1_ref[...], 0.0)
        o = jnp.maximum(
            jnp.dot(o, v2_ref[...], preferred_element_type=jnp.float32)
            + c2_ref[...], 0.0)
        o_ref[...] = (jnp.dot(o, v3_ref[...], preferred_element_type=jnp.float32)
                      + c3_ref[...])

    return pl.pallas_call(
        body,
        out_shape=jax.ShapeDtypeStruct((NB, OUT), jnp.float32),
    )(s_p, c_p, batch_pad, u, gamma_u.reshape(1, G), beta_u.reshape(1, G),
      V1, c1.reshape(1, BIGGER), V2, c2.reshape(1, BIGGER),
      V3, c3.reshape(1, OUT))


def kernel(x, edge_index, u, batch, gamma_x, beta_x, gamma_u, beta_u,
           W1, b1, W2, b2, W3, b3, V1, c1, V2, c2, V3, c3):
    src_r = edge_index[0].reshape(NW, K, C)
    dst_r = edge_index[1].reshape(NW, K, C)
    A, B = _tc_prep(x, gamma_x, beta_x, W1)
    gA, gB = _sc_gather(A, B, dst_r, src_r)
    eye4 = jnp.eye(4, dtype=jnp.float32)
    h3p = _tc_mlp(gA.reshape(E4, 128), gB.reshape(E4, 128),
                  jnp.tile(b1, 4).reshape(1, 128),
                  jnp.kron(eye4, W2), jnp.tile(b2, 4).reshape(1, 128),
                  jnp.kron(eye4, W3), jnp.tile(b3, 4).reshape(1, 128))
    s_p, c_p = _sc_scatter(h3p.reshape(E, H), dst_r,
                           jnp.zeros((N_PAD, H), jnp.float32),
                           jnp.zeros((N_PAD, CW), jnp.float32),
                           jnp.ones((C, CW), jnp.float32))
    batch_pad = jnp.concatenate(
        [batch, jnp.full((N_PAD - N,), NB, jnp.int32)]).reshape(1, N_PAD)
    return _tc_final(s_p, c_p, batch_pad, u, gamma_u, beta_u,
                     V1, c1, V2, c2, V3, c3)


# speedup vs baseline: 14.0578x; 1.0310x over previous
import functools

import jax
import jax.numpy as jnp
from jax import lax
from jax.experimental import pallas as pl
from jax.experimental.pallas import tpu as pltpu
from jax.experimental.pallas import tpu_sc as plsc

N = 10000
E = 320000
D = 128
H = 32
NB = 64
G = 2
BIGGER = 128
OUT = 1

NW = 32
EW = E // NW
C = 500
K = EW // C

N_PAD = 10240
RPT = N_PAD // 16
CW = 16

_MESH = dict(core_axis_name="c", subcore_axis_name="s")


def _tc_prep(x, gamma, beta, W1):
    def body(x_ref, g_ref, b_ref, w_ref, t_ref):
        xv = x_ref[...]
        m = jnp.mean(xv, axis=0, keepdims=True)
        v = jnp.mean((xv - m) ** 2, axis=0, keepdims=True)
        xn = (xv - m) * lax.rsqrt(v + 1e-5) * g_ref[...] + b_ref[...]
        wa = w_ref[:D, :]
        wb = w_ref[D:, :]
        a = jnp.dot(xn, wa - wb, preferred_element_type=jnp.float32)
        bb = jnp.dot(xn, wb, preferred_element_type=jnp.float32)
        t_ref[...] = jnp.concatenate(
            [a, bb, jnp.zeros((N, D - 2 * H), jnp.float32)], axis=1)

    return pl.pallas_call(
        body,
        out_shape=jax.ShapeDtypeStruct((N, D), jnp.float32),
    )(x, gamma.reshape(1, D), beta.reshape(1, D), W1)


def _sc_gather(T, dst_r, src_r):
    @functools.partial(
        pl.kernel,
        out_type=(jax.ShapeDtypeStruct((E, H), jnp.float32),
                  jax.ShapeDtypeStruct((E, H), jnp.float32)),
        mesh=plsc.VectorSubcoreMesh(**_MESH),
        compiler_params=pltpu.CompilerParams(use_tc_tiling_on_sc=False),
        scratch_types=[
            pltpu.VMEM((K, C), jnp.int32),
            pltpu.VMEM((K, C), jnp.int32),
            pltpu.VMEM((C, H), jnp.float32),
            pltpu.VMEM((C, H), jnp.float32),
            pltpu.VMEM((C, H), jnp.float32),
            pltpu.VMEM((C, H), jnp.float32),
            pltpu.VMEM_SHARED((N, H), jnp.float32),
            pltpu.VMEM_SHARED((N, H), jnp.float32),
            pltpu.SemaphoreType.DMA,
            pltpu.SemaphoreType.DMA,
            pltpu.SemaphoreType.DMA,
            pltpu.SemaphoreType.DMA,
            pltpu.SemaphoreType.DMA,
            pltpu.SemaphoreType.DMA,
            pltpu.SemaphoreType.DMA,
            pltpu.SemaphoreType.DMA,
        ],
    )
    def k(t_hbm, dst_hbm, src_hbm, ga_hbm, gb_hbm,
          dst_v, src_v, buf_a0, buf_a1, buf_b0, buf_b1, a_sh, b_sh,
          sem_a0, sem_a1, sem_b0, sem_b1, osem_a0, osem_a1, osem_b0,
          osem_b1):
        sid = lax.axis_index("s")
        wid = lax.axis_index("c") * 16 + sid
        pltpu.sync_copy(dst_hbm.at[wid], dst_v)
        pltpu.sync_copy(src_hbm.at[wid], src_v)

        @pl.when(sid == 0)
        def _():
            pltpu.sync_copy(t_hbm.at[:, pl.ds(0, H)], a_sh)
            pltpu.sync_copy(t_hbm.at[:, pl.ds(H, H)], b_sh)

        plsc.subcore_barrier()
        base = wid * EW
        bufs_a = (buf_a0, buf_a1)
        bufs_b = (buf_b0, buf_b1)
        sems_a = (sem_a0, sem_a1)
        sems_b = (sem_b0, sem_b1)

        def start_g(j, b):
            pltpu.async_copy(a_sh.at[dst_v.at[j]], bufs_a[b], sems_a[b])
            pltpu.async_copy(b_sh.at[src_v.at[j]], bufs_b[b], sems_b[b])

        def wait_g(j, b):
            pltpu.make_async_copy(a_sh.at[dst_v.at[j]], bufs_a[b],
                                  sems_a[b]).wait()
            pltpu.make_async_copy(b_sh.at[src_v.at[j]], bufs_b[b],
                                  sems_b[b]).wait()

        osems_a = (osem_a0, osem_a1)
        osems_b = (osem_b0, osem_b1)

        def start_o(j, b):
            pltpu.async_copy(bufs_a[b], ga_hbm.at[pl.ds(base + j * C, C)],
                             osems_a[b])
            pltpu.async_copy(bufs_b[b], gb_hbm.at[pl.ds(base + j * C, C)],
                             osems_b[b])

        def wait_o(j, b):
            pltpu.make_async_copy(bufs_a[b],
                                  ga_hbm.at[pl.ds(base + j * C, C)],
                                  osems_a[b]).wait()
            pltpu.make_async_copy(bufs_b[b],
                                  gb_hbm.at[pl.ds(base + j * C, C)],
                                  osems_b[b]).wait()

        start_g(0, 0)
        wait_g(0, 0)
        start_o(0, 0)
        start_g(1, 1)

        def step(j, carry):
            for b in (0, 1):
                @pl.when(j % 2 == b)
                def _():
                    wait_g(j, b)
                    start_o(j, b)
                    wait_o(j - 1, 1 - b)
                    start_g(j + 1, 1 - b)
            return carry

        lax.fori_loop(1, K - 1, step, 0)
        bl = (K - 1) % 2
        wait_g(K - 1, bl)
        start_o(K - 1, bl)
        wait_o(K - 2, 1 - bl)
        wait_o(K - 1, bl)

    return k(T, dst_r, src_r)


E4 = E // 4


def _tc_mlp(gA4, gB4, b1t, W2blk, b2t, W3blk, b3t):
    BE4 = 2000

    def body(a_ref, g_ref, b1_ref, w2_ref, b2_ref, w3_ref, b3_ref, o_ref):
        h = jnp.maximum(a_ref[...] + g_ref[...] + b1_ref[...], 0.0)
        h = jnp.dot(h, w2_ref[...], preferred_element_type=jnp.float32)
        h = jnp.maximum(h + b2_ref[...], 0.0)
        h = jnp.dot(h, w3_ref[...], preferred_element_type=jnp.float32)
        o_ref[...] = jnp.maximum(h + b3_ref[...], 0.0)

    full = lambda i: (0, 0)
    return pl.pallas_call(
        body,
        grid=(E4 // BE4,),
        in_specs=[
            pl.BlockSpec((BE4, 128), lambda i: (i, 0)),
            pl.BlockSpec((BE4, 128), lambda i: (i, 0)),
            pl.BlockSpec((1, 128), full),
            pl.BlockSpec((128, 128), lambda i: (0, 0)),
            pl.BlockSpec((1, 128), full),
            pl.BlockSpec((128, 128), lambda i: (0, 0)),
            pl.BlockSpec((1, 128), full),
        ],
        out_specs=pl.BlockSpec((BE4, 128), lambda i: (i, 0)),
        out_shape=jax.ShapeDtypeStruct((E4, 128), jnp.float32),
    )(gA4, gB4, b1t, W2blk, b2t, W3blk, b3t)


def _sc_scatter(h3, dst_r, s_init, c_init, ones):
    @functools.partial(
        pl.kernel,
        out_type=(jax.ShapeDtypeStruct((2, N_PAD, H), jnp.float32),
                  jax.ShapeDtypeStruct((2, N_PAD, CW), jnp.float32)),
        mesh=plsc.VectorSubcoreMesh(**_MESH),
        compiler_params=pltpu.CompilerParams(use_tc_tiling_on_sc=False),
        scratch_types=[
            pltpu.VMEM((K, C), jnp.int32),
            pltpu.VMEM((C, H), jnp.float32),
            pltpu.VMEM((C, H), jnp.float32),
            pltpu.VMEM((C, CW), jnp.float32),
            pltpu.VMEM_SHARED((N_PAD, H), jnp.float32),
            pltpu.VMEM_SHARED((N_PAD, CW), jnp.float32),
            pltpu.SemaphoreType.DMA,
            pltpu.SemaphoreType.DMA,
            pltpu.SemaphoreType.DMA,
            pltpu.SemaphoreType.DMA,
            pltpu.SemaphoreType.DMA,
            pltpu.SemaphoreType.DMA,
        ],
    )
    def k(h3_hbm, dst_hbm, sinit_hbm, cinit_hbm, ones_hbm, s_hbm, cnt_hbm,
          dst_v, hbuf0, hbuf1, ones_v, s_acc, c_acc, sem0, sem1,
          ssem0, ssem1, csem0, csem1):
        cid = lax.axis_index("c")
        sid = lax.axis_index("s")
        wid = cid * 16 + sid
        pltpu.sync_copy(dst_hbm.at[wid], dst_v)
        pltpu.sync_copy(ones_hbm, ones_v)

        slab = pl.ds(sid * RPT, RPT)
        pltpu.sync_copy(sinit_hbm.at[slab], s_acc.at[slab])
        pltpu.sync_copy(cinit_hbm.at[slab], c_acc.at[slab])
        plsc.subcore_barrier()

        base = wid * EW
        hbufs = (hbuf0, hbuf1)
        sems = (sem0, sem1)

        def start_l(j, b):
            pltpu.async_copy(h3_hbm.at[pl.ds(base + j * C, C)], hbufs[b],
                             sems[b])

        def wait_l(j, b):
            pltpu.make_async_copy(h3_hbm.at[pl.ds(base + j * C, C)],
                                  hbufs[b], sems[b]).wait()

        ssems = (ssem0, ssem1)
        csems = (csem0, csem1)

        def start_s(j, b):
            pltpu.async_copy(hbufs[b], s_acc.at[dst_v.at[j]], ssems[b],
                             add=True)
            pltpu.async_copy(ones_v, c_acc.at[dst_v.at[j]], csems[b],
                             add=True)

        def wait_s(j, b):
            pltpu.make_async_copy(hbufs[b], s_acc.at[dst_v.at[j]],
                                  ssems[b]).wait()
            pltpu.make_async_copy(ones_v, c_acc.at[dst_v.at[j]],
                                  csems[b]).wait()

        start_l(0, 0)
        wait_l(0, 0)
        start_s(0, 0)
        start_l(1, 1)

        def step(j, carry):
            for b in (0, 1):
                @pl.when(j % 2 == b)
                def _():
                    wait_l(j, b)
                    start_s(j, b)
                    wait_s(j - 1, 1 - b)
                    start_l(j + 1, 1 - b)
            return carry

        lax.fori_loop(1, K - 1, step, 0)
        bl = (K - 1) % 2
        wait_l(K - 1, bl)
        start_s(K - 1, bl)
        wait_s(K - 2, 1 - bl)
        wait_s(K - 1, bl)
        plsc.subcore_barrier()

        pltpu.sync_copy(s_acc.at[slab], s_hbm.at[cid, slab])
        pltpu.sync_copy(c_acc.at[slab], cnt_hbm.at[cid, slab])

    return k(h3, dst_r, s_init, c_init, ones)


def _tc_final(s_p, c_p, batch_pad, u, gamma_u, beta_u, V1, c1, V2, c2, V3, c3):
    def body(s_ref, c_ref, bt_ref, u_ref, gu_ref, bu_ref,
             v1_ref, c1_ref, v2_ref, c2_ref, v3_ref, c3_ref, o_ref):
        s = s_ref[0] + s_ref[1]
        cnt = c_ref[0, :, :1] + c_ref[1, :, :1]
        xc = s / jnp.maximum(cnt, 1.0)
        b = bt_ref[...]
        oh = (lax.broadcasted_iota(jnp.int32, (NB, N_PAD), 0) == b
              ).astype(jnp.float32)
        gs = jnp.dot(oh, xc, preferred_element_type=jnp.float32)
        gc = jnp.sum(oh, axis=1, keepdims=True)
        u2 = gs / jnp.maximum(gc, 1.0)
        uv = u_ref[...]
        mu = jnp.mean(uv, axis=0, keepdims=True)
        vu = jnp.mean((uv - mu) ** 2, axis=0, keepdims=True)
        u1 = (uv - mu) * lax.rsqrt(vu + 1e-5) * gu_ref[...] + bu_ref[...]
        uc = jnp.concatenate([u1, u2], axis=1)
        o = jnp.maximum(
            jnp.dot(uc, v1_ref[...], preferred_element_type=jnp.float32)
            + c1_ref[...], 0.0)
        o = jnp.maximum(
            jnp.dot(o, v2_ref[...], preferred_element_type=jnp.float32)
            + c2_ref[...], 0.0)
        o_ref[...] = (jnp.dot(o, v3_ref[...], preferred_element_type=jnp.float32)
                      + c3_ref[...])

    return pl.pallas_call(
        body,
        out_shape=jax.ShapeDtypeStruct((NB, OUT), jnp.float32),
    )(s_p, c_p, batch_pad, u, gamma_u.reshape(1, G), beta_u.reshape(1, G),
      V1, c1.reshape(1, BIGGER), V2, c2.reshape(1, BIGGER),
      V3, c3.reshape(1, OUT))


def kernel(x, edge_index, u, batch, gamma_x, beta_x, gamma_u, beta_u,
           W1, b1, W2, b2, W3, b3, V1, c1, V2, c2, V3, c3):
    src_r = edge_index[0].reshape(NW, K, C)
    dst_r = edge_index[1].reshape(NW, K, C)
    T = _tc_prep(x, gamma_x, beta_x, W1)
    gA, gB = _sc_gather(T, dst_r, src_r)
    eye4 = jnp.eye(4, dtype=jnp.float32)
    h3p = _tc_mlp(gA.reshape(E4, 128), gB.reshape(E4, 128),
                  jnp.tile(b1, 4).reshape(1, 128),
                  jnp.kron(eye4, W2), jnp.tile(b2, 4).reshape(1, 128),
                  jnp.kron(eye4, W3), jnp.tile(b3, 4).reshape(1, 128))
    s_p, c_p = _sc_scatter(h3p.reshape(E, H), dst_r,
                           jnp.zeros((N_PAD, H), jnp.float32),
                           jnp.zeros((N_PAD, CW), jnp.float32),
                           jnp.ones((C, CW), jnp.float32))
    batch_pad = jnp.concatenate(
        [batch, jnp.full((N_PAD - N,), NB, jnp.int32)]).reshape(1, N_PAD)
    return _tc_final(s_p, c_p, batch_pad, u, gamma_u, beta_u,
                     V1, c1, V2, c2, V3, c3)


# speedup vs baseline: 15.1348x; 1.0766x over previous
import functools

import jax
import jax.numpy as jnp
from jax import lax
from jax.experimental import pallas as pl
from jax.experimental.pallas import tpu as pltpu
from jax.experimental.pallas import tpu_sc as plsc

N = 10000
E = 320000
D = 128
H = 32
NB = 64
G = 2
BIGGER = 128
OUT = 1

NW = 32
EW = E // NW
C = 500
K = EW // C

N_PAD = 10240
RPT = N_PAD // 16
CW = 16

_MESH = dict(core_axis_name="c", subcore_axis_name="s")


def _tc_prep(x, gamma, beta, W1):
    def body(x_ref, g_ref, b_ref, w_ref, t_ref):
        xv = x_ref[...]
        m = jnp.mean(xv, axis=0, keepdims=True)
        v = jnp.mean((xv - m) ** 2, axis=0, keepdims=True)
        xn = (xv - m) * lax.rsqrt(v + 1e-5) * g_ref[...] + b_ref[...]
        wa = w_ref[:D, :]
        wb = w_ref[D:, :]
        a = jnp.dot(xn, wa - wb, preferred_element_type=jnp.float32)
        bb = jnp.dot(xn, wb, preferred_element_type=jnp.float32)
        t_ref[...] = jnp.concatenate(
            [a, bb, jnp.zeros((N, D - 2 * H), jnp.float32)], axis=1)

    return pl.pallas_call(
        body,
        out_shape=jax.ShapeDtypeStruct((N, D), jnp.float32),
    )(x, gamma.reshape(1, D), beta.reshape(1, D), W1)


def _sc_gather(T, dst_r, src_r):
    @functools.partial(
        pl.kernel,
        out_type=(jax.ShapeDtypeStruct((E, H), jnp.float32),
                  jax.ShapeDtypeStruct((E, H), jnp.float32)),
        mesh=plsc.VectorSubcoreMesh(**_MESH),
        compiler_params=pltpu.CompilerParams(use_tc_tiling_on_sc=False),
        scratch_types=[
            pltpu.VMEM((K, C), jnp.int32),
            pltpu.VMEM((K, C), jnp.int32),
            pltpu.VMEM((C, H), jnp.float32),
            pltpu.VMEM((C, H), jnp.float32),
            pltpu.VMEM((C, H), jnp.float32),
            pltpu.VMEM((C, H), jnp.float32),
            pltpu.VMEM_SHARED((N, H), jnp.float32),
            pltpu.VMEM_SHARED((N, H), jnp.float32),
            pltpu.SemaphoreType.DMA,
            pltpu.SemaphoreType.DMA,
            pltpu.SemaphoreType.DMA,
            pltpu.SemaphoreType.DMA,
            pltpu.SemaphoreType.DMA,
            pltpu.SemaphoreType.DMA,
            pltpu.SemaphoreType.DMA,
            pltpu.SemaphoreType.DMA,
        ],
    )
    def k(t_hbm, dst_hbm, src_hbm, ga_hbm, gb_hbm,
          dst_v, src_v, buf_a0, buf_a1, buf_b0, buf_b1, a_sh, b_sh,
          sem_a0, sem_a1, sem_b0, sem_b1, osem_a0, osem_a1, osem_b0,
          osem_b1):
        sid = lax.axis_index("s")
        wid = lax.axis_index("c") * 16 + sid
        pltpu.sync_copy(dst_hbm.at[wid], dst_v)
        pltpu.sync_copy(src_hbm.at[wid], src_v)

        @pl.when(sid == 0)
        def _():
            pltpu.sync_copy(t_hbm.at[:, pl.ds(0, H)], a_sh)
            pltpu.sync_copy(t_hbm.at[:, pl.ds(H, H)], b_sh)

        plsc.subcore_barrier()
        base = wid * EW
        bufs_a = (buf_a0, buf_a1)
        bufs_b = (buf_b0, buf_b1)
        sems_a = (sem_a0, sem_a1)
        sems_b = (sem_b0, sem_b1)

        def start_g(j, b):
            pltpu.async_copy(a_sh.at[dst_v.at[j]], bufs_a[b], sems_a[b])
            pltpu.async_copy(b_sh.at[src_v.at[j]], bufs_b[b], sems_b[b])

        def wait_g(j, b):
            pltpu.make_async_copy(a_sh.at[dst_v.at[j]], bufs_a[b],
                                  sems_a[b]).wait()
            pltpu.make_async_copy(b_sh.at[src_v.at[j]], bufs_b[b],
                                  sems_b[b]).wait()

        osems_a = (osem_a0, osem_a1)
        osems_b = (osem_b0, osem_b1)

        def start_o(j, b):
            pltpu.async_copy(bufs_a[b], ga_hbm.at[pl.ds(base + j * C, C)],
                             osems_a[b])
            pltpu.async_copy(bufs_b[b], gb_hbm.at[pl.ds(base + j * C, C)],
                             osems_b[b])

        def wait_o(j, b):
            pltpu.make_async_copy(bufs_a[b],
                                  ga_hbm.at[pl.ds(base + j * C, C)],
                                  osems_a[b]).wait()
            pltpu.make_async_copy(bufs_b[b],
                                  gb_hbm.at[pl.ds(base + j * C, C)],
                                  osems_b[b]).wait()

        start_g(0, 0)
        wait_g(0, 0)
        start_o(0, 0)
        start_g(1, 1)

        def step(j, carry):
            for b in (0, 1):
                @pl.when(j % 2 == b)
                def _():
                    wait_g(j, b)
                    start_o(j, b)
                    wait_o(j - 1, 1 - b)
                    start_g(j + 1, 1 - b)
            return carry

        lax.fori_loop(1, K - 1, step, 0)
        bl = (K - 1) % 2
        wait_g(K - 1, bl)
        start_o(K - 1, bl)
        wait_o(K - 2, 1 - bl)
        wait_o(K - 1, bl)

    return k(T, dst_r, src_r)


E4 = E // 4


def _tc_mlp(gA4, gB4, b1t, W2blk, b2t, W3blk, b3t):
    BE4 = 8000

    def body(a_ref, g_ref, b1_ref, w2_ref, b2_ref, w3_ref, b3_ref, o_ref):
        h = jnp.maximum(a_ref[...] + g_ref[...] + b1_ref[...], 0.0)
        h = jnp.dot(h, w2_ref[...], preferred_element_type=jnp.float32)
        h = jnp.maximum(h + b2_ref[...], 0.0)
        h = jnp.dot(h, w3_ref[...], preferred_element_type=jnp.float32)
        o_ref[...] = jnp.maximum(h + b3_ref[...], 0.0)

    full = lambda i: (0, 0)
    return pl.pallas_call(
        body,
        grid=(E4 // BE4,),
        in_specs=[
            pl.BlockSpec((BE4, 128), lambda i: (i, 0)),
            pl.BlockSpec((BE4, 128), lambda i: (i, 0)),
            pl.BlockSpec((1, 128), full),
            pl.BlockSpec((128, 128), lambda i: (0, 0)),
            pl.BlockSpec((1, 128), full),
            pl.BlockSpec((128, 128), lambda i: (0, 0)),
            pl.BlockSpec((1, 128), full),
        ],
        out_specs=pl.BlockSpec((BE4, 128), lambda i: (i, 0)),
        out_shape=jax.ShapeDtypeStruct((E4, 128), jnp.float32),
    )(gA4, gB4, b1t, W2blk, b2t, W3blk, b3t)


def _sc_scatter(h3, dst_r, s_init, c_init, ones):
    @functools.partial(
        pl.kernel,
        out_type=(jax.ShapeDtypeStruct((2, N_PAD, H), jnp.float32),
                  jax.ShapeDtypeStruct((2, N_PAD, CW), jnp.float32)),
        mesh=plsc.VectorSubcoreMesh(**_MESH),
        compiler_params=pltpu.CompilerParams(use_tc_tiling_on_sc=False),
        scratch_types=[
            pltpu.VMEM((K, C), jnp.int32),
            pltpu.VMEM((C, H), jnp.float32),
            pltpu.VMEM((C, H), jnp.float32),
            pltpu.VMEM((C, CW), jnp.float32),
            pltpu.VMEM_SHARED((N_PAD, H), jnp.float32),
            pltpu.VMEM_SHARED((N_PAD, CW), jnp.float32),
            pltpu.SemaphoreType.DMA,
            pltpu.SemaphoreType.DMA,
            pltpu.SemaphoreType.DMA,
            pltpu.SemaphoreType.DMA,
            pltpu.SemaphoreType.DMA,
            pltpu.SemaphoreType.DMA,
        ],
    )
    def k(h3_hbm, dst_hbm, sinit_hbm, cinit_hbm, ones_hbm, s_hbm, cnt_hbm,
          dst_v, hbuf0, hbuf1, ones_v, s_acc, c_acc, sem0, sem1,
          ssem0, ssem1, csem0, csem1):
        cid = lax.axis_index("c")
        sid = lax.axis_index("s")
        wid = cid * 16 + sid
        pltpu.sync_copy(dst_hbm.at[wid], dst_v)
        pltpu.sync_copy(ones_hbm, ones_v)

        slab = pl.ds(sid * RPT, RPT)
        pltpu.sync_copy(sinit_hbm.at[slab], s_acc.at[slab])
        pltpu.sync_copy(cinit_hbm.at[slab], c_acc.at[slab])
        plsc.subcore_barrier()

        base = wid * EW
        hbufs = (hbuf0, hbuf1)
        sems = (sem0, sem1)

        def start_l(j, b):
            pltpu.async_copy(h3_hbm.at[pl.ds(base + j * C, C)], hbufs[b],
                             sems[b])

        def wait_l(j, b):
            pltpu.make_async_copy(h3_hbm.at[pl.ds(base + j * C, C)],
                                  hbufs[b], sems[b]).wait()

        ssems = (ssem0, ssem1)
        csems = (csem0, csem1)

        def start_s(j, b):
            pltpu.async_copy(hbufs[b], s_acc.at[dst_v.at[j]], ssems[b],
                             add=True)
            pltpu.async_copy(ones_v, c_acc.at[dst_v.at[j]], csems[b],
                             add=True)

        def wait_s(j, b):
            pltpu.make_async_copy(hbufs[b], s_acc.at[dst_v.at[j]],
                                  ssems[b]).wait()
            pltpu.make_async_copy(ones_v, c_acc.at[dst_v.at[j]],
                                  csems[b]).wait()

        start_l(0, 0)
        wait_l(0, 0)
        start_s(0, 0)
        start_l(1, 1)

        def step(j, carry):
            for b in (0, 1):
                @pl.when(j % 2 == b)
                def _():
                    wait_l(j, b)
                    start_s(j, b)
                    wait_s(j - 1, 1 - b)
                    start_l(j + 1, 1 - b)
            return carry

        lax.fori_loop(1, K - 1, step, 0)
        bl = (K - 1) % 2
        wait_l(K - 1, bl)
        start_s(K - 1, bl)
        wait_s(K - 2, 1 - bl)
        wait_s(K - 1, bl)
        plsc.subcore_barrier()

        pltpu.sync_copy(s_acc.at[slab], s_hbm.at[cid, slab])
        pltpu.sync_copy(c_acc.at[slab], cnt_hbm.at[cid, slab])

    return k(h3, dst_r, s_init, c_init, ones)


def _tc_final(s_p, c_p, batch_pad, u, gamma_u, beta_u, V1, c1, V2, c2, V3, c3):
    def body(s_ref, c_ref, bt_ref, u_ref, gu_ref, bu_ref,
             v1_ref, c1_ref, v2_ref, c2_ref, v3_ref, c3_ref, o_ref):
        s = s_ref[0] + s_ref[1]
        cnt = c_ref[0, :, :1] + c_ref[1, :, :1]
        xc = s / jnp.maximum(cnt, 1.0)
        b = bt_ref[...]
        oh = (lax.broadcasted_iota(jnp.int32, (NB, N_PAD), 0) == b
              ).astype(jnp.float32)
        gs = jnp.dot(oh, xc, preferred_element_type=jnp.float32)
        gc = jnp.sum(oh, axis=1, keepdims=True)
        u2 = gs / jnp.maximum(gc, 1.0)
        uv = u_ref[...]
        mu = jnp.mean(uv, axis=0, keepdims=True)
        vu = jnp.mean((uv - mu) ** 2, axis=0, keepdims=True)
        u1 = (uv - mu) * lax.rsqrt(vu + 1e-5) * gu_ref[...] + bu_ref[...]
        uc = jnp.concatenate([u1, u2], axis=1)
        o = jnp.maximum(
            jnp.dot(uc, v1_ref[...], preferred_element_type=jnp.float32)
            + c1_ref[...], 0.0)
        o = jnp.maximum(
            jnp.dot(o, v2_ref[...], preferred_element_type=jnp.float32)
            + c2_ref[...], 0.0)
        o_ref[...] = (jnp.dot(o, v3_ref[...], preferred_element_type=jnp.float32)
                      + c3_ref[...])

    return pl.pallas_call(
        body,
        out_shape=jax.ShapeDtypeStruct((NB, OUT), jnp.float32),
    )(s_p, c_p, batch_pad, u, gamma_u.reshape(1, G), beta_u.reshape(1, G),
      V1, c1.reshape(1, BIGGER), V2, c2.reshape(1, BIGGER),
      V3, c3.reshape(1, OUT))


def kernel(x, edge_index, u, batch, gamma_x, beta_x, gamma_u, beta_u,
           W1, b1, W2, b2, W3, b3, V1, c1, V2, c2, V3, c3):
    src_r = edge_index[0].reshape(NW, K, C)
    dst_r = edge_index[1].reshape(NW, K, C)
    T = _tc_prep(x, gamma_x, beta_x, W1)
    gA, gB = _sc_gather(T, dst_r, src_r)
    eye4 = jnp.eye(4, dtype=jnp.float32)
    h3p = _tc_mlp(gA.reshape(E4, 128), gB.reshape(E4, 128),
                  jnp.tile(b1, 4).reshape(1, 128),
                  jnp.kron(eye4, W2), jnp.tile(b2, 4).reshape(1, 128),
                  jnp.kron(eye4, W3), jnp.tile(b3, 4).reshape(1, 128))
    s_p, c_p = _sc_scatter(h3p.reshape(E, H), dst_r,
                           jnp.zeros((N_PAD, H), jnp.float32),
                           jnp.zeros((N_PAD, CW), jnp.float32),
                           jnp.ones((C, CW), jnp.float32))
    batch_pad = jnp.concatenate(
        [batch, jnp.full((N_PAD - N,), NB, jnp.int32)]).reshape(1, N_PAD)
    return _tc_final(s_p, c_p, batch_pad, u, gamma_u, beta_u,
                     V1, c1, V2, c2, V3, c3)
